# Initial kernel scaffold; baseline (speedup 1.0000x reference)
import jax
import jax.numpy as jnp
from jax.experimental import pallas as pl


def kernel(x, mask, table, W_pnnn, b_pnnn, W1, b1, W2, b2, Wmu, bmu, Wlv, blv, eps):
    raise NotImplementedError("write your pallas kernel here")


# speedup vs baseline: 2.9459x; 2.9459x over previous
import functools

import jax
import jax.numpy as jnp
from jax import lax
from jax.experimental import pallas as pl
from jax.experimental.pallas import tpu as pltpu
from jax.experimental.pallas import tpu_sc as plsc

B, L = 1024, 100
EMB = 16
K_DIM, H1, H2, LAT = 64, 128, 64, 32

NC, NS = 2, 16
NW = NC * NS
TOT = B * L
PER_W = TOT // NW
CHUNK = 128
NCH = PER_W // CHUNK

BB = 64
GRID = B // BB


def _sc_gather(x32, table):
    mesh = plsc.VectorSubcoreMesh(core_axis_name="c", subcore_axis_name="s")

    @functools.partial(
        pl.kernel,
        out_type=jax.ShapeDtypeStruct((NW, PER_W, EMB), jnp.float32),
        mesh=mesh,
        scratch_types=[
            pltpu.VMEM((NCH, CHUNK), jnp.int32),
            pltpu.VMEM((PER_W, EMB), jnp.float32),
            pltpu.SemaphoreType.DMA,
        ],
        compiler_params=pltpu.CompilerParams(use_tc_tiling_on_sc=False),
    )
    def gather_kernel(x_hbm, table_hbm, out_hbm, idx_v, rows_v, sem):
        wid = lax.axis_index("s") * NC + lax.axis_index("c")
        pltpu.sync_copy(x_hbm.at[wid], idx_v)

        def fire(j, _):
            pltpu.make_async_copy(
                table_hbm.at[idx_v.at[j]],
                rows_v.at[pl.ds(j * CHUNK, CHUNK)],
                sem,
            ).start()
            return _

        lax.fori_loop(0, NCH, fire, 0, unroll=False)

        def drain(j, _):
            pltpu.make_async_copy(
                table_hbm.at[idx_v.at[j]],
                rows_v.at[pl.ds(j * CHUNK, CHUNK)],
                sem,
            ).wait()
            return _

        lax.fori_loop(0, NCH, drain, 0, unroll=False)
        pltpu.sync_copy(rows_v, out_hbm.at[wid])

    return gather_kernel(x32, table)


def _tc_fused(e, mask_f, wp, bp, w1, b1, w2, b2, wmu, bmu, wlv, blv, eps):
    def body(e_ref, m_ref, wp_ref, bp_ref, w1_ref, b1_ref, w2_ref, b2_ref,
             wmu_ref, bmu_ref, wlv_ref, blv_ref, eps_ref,
             z_ref, mu_ref, lv_ref):
        hp = jax.lax.Precision.HIGHEST
        ev = e_ref[...]
        p = jnp.dot(ev, wp_ref[...], precision=hp) + bp_ref[...]
        p = jnp.maximum(p, 0.0)
        p = p.reshape(BB, L, K_DIM) * m_ref[...][:, :, None]
        pnc = jnp.sum(p, axis=1)
        h = jnp.maximum(jnp.dot(pnc, w1_ref[...], precision=hp) + b1_ref[...], 0.0)
        h = jnp.maximum(jnp.dot(h, w2_ref[...], precision=hp) + b2_ref[...], 0.0)
        mu = jnp.dot(h, wmu_ref[...], precision=hp) + bmu_ref[...]
        lv = jnp.dot(h, wlv_ref[...], precision=hp) + blv_ref[...]
        z = mu + eps_ref[...] * jnp.exp(0.5 * lv)
        z_ref[...] = z
        mu_ref[...] = mu
        lv_ref[...] = lv

    rep = lambda shape: pl.BlockSpec(shape, lambda i: (0,) * len(shape))
    out_sds = jax.ShapeDtypeStruct((B, LAT), jnp.float32)
    return pl.pallas_call(
        body,
        grid=(GRID,),
        in_specs=[
            pl.BlockSpec((BB * L, EMB), lambda i: (i, 0)),
            pl.BlockSpec((BB, L), lambda i: (i, 0)),
            rep((EMB, K_DIM)), rep((1, K_DIM)),
            rep((K_DIM, H1)), rep((1, H1)),
            rep((H1, H2)), rep((1, H2)),
            rep((H2, LAT)), rep((1, LAT)),
            rep((H2, LAT)), rep((1, LAT)),
            pl.BlockSpec((BB, LAT), lambda i: (i, 0)),
        ],
        out_specs=[
            pl.BlockSpec((BB, LAT), lambda i: (i, 0)),
            pl.BlockSpec((BB, LAT), lambda i: (i, 0)),
            pl.BlockSpec((BB, LAT), lambda i: (i, 0)),
        ],
        out_shape=[out_sds, out_sds, out_sds],
    )(e, mask_f, wp, bp, w1, b1, w2, b2, wmu, bmu, wlv, blv, eps)


def kernel(x, mask, table, W_pnnn, b_pnnn, W1, b1, W2, b2, Wmu, bmu, Wlv, blv, eps):
    x32 = x.reshape(NW, NCH, CHUNK)
    gathered = _sc_gather(x32, table).reshape(TOT, EMB)
    mask_f = mask.astype(jnp.float32)
    z, mu, lv = _tc_fused(
        gathered, mask_f,
        W_pnnn.T, b_pnnn.reshape(1, K_DIM),
        W1.T, b1.reshape(1, H1),
        W2.T, b2.reshape(1, H2),
        Wmu.T, bmu.reshape(1, LAT),
        Wlv.T, blv.reshape(1, LAT),
        eps,
    )
    return (z, mu, lv)


# speedup vs baseline: 3.8286x; 1.2996x over previous
import functools

import jax
import jax.numpy as jnp
from jax import lax
from jax.experimental import pallas as pl
from jax.experimental.pallas import tpu as pltpu
from jax.experimental.pallas import tpu_sc as plsc

B, L = 1024, 100
EMB = 16
K_DIM, H1, H2, LAT = 64, 128, 64, 32

NC, NS = 2, 16
NW = NC * NS
TOT = B * L
PER_W = TOT // NW
CHUNK = 128
NCH = PER_W // CHUNK

BB = 64
GRID = B // BB

V = 1000000
CC = 8192


def _tc_pack(table_t):
    def body(i_ref, o_ref):
        t = i_ref[...].T
        t3 = t.reshape(CC // 8, 8, 16)
        o_ref[...] = jnp.concatenate([t3[:, j, :] for j in range(8)], axis=1)

    return pl.pallas_call(
        body,
        grid=((V + CC - 1) // CC,),
        in_specs=[pl.BlockSpec((16, CC), lambda i: (0, i))],
        out_specs=pl.BlockSpec((CC // 8, 128), lambda i: (i, 0)),
        out_shape=jax.ShapeDtypeStruct((V // 8, 128), jnp.float32),
    )(table_t)


def _sc_gather(x32, table):
    mesh = plsc.VectorSubcoreMesh(core_axis_name="c", subcore_axis_name="s")

    @functools.partial(
        pl.kernel,
        out_type=jax.ShapeDtypeStruct((NW, PER_W, EMB), jnp.float32),
        mesh=mesh,
        scratch_types=[
            pltpu.VMEM((NCH, CHUNK), jnp.int32),
            pltpu.VMEM((PER_W, EMB), jnp.float32),
            pltpu.SemaphoreType.DMA,
        ],
        compiler_params=pltpu.CompilerParams(use_tc_tiling_on_sc=False),
    )
    def gather_kernel(x_hbm, table_hbm, out_hbm, idx_v, rows_v, sem):
        wid = lax.axis_index("s") * NC + lax.axis_index("c")
        pltpu.sync_copy(x_hbm.at[wid], idx_v)

        def fire(j, _):
            pltpu.make_async_copy(
                table_hbm.at[idx_v.at[j]],
                rows_v.at[pl.ds(j * CHUNK, CHUNK)],
                sem,
            ).start()
            return _

        lax.fori_loop(0, NCH, fire, 0, unroll=False)

        def drain(j, _):
            pltpu.make_async_copy(
                table_hbm.at[idx_v.at[j]],
                rows_v.at[pl.ds(j * CHUNK, CHUNK)],
                sem,
            ).wait()
            return _

        lax.fori_loop(0, NCH, drain, 0, unroll=False)
        pltpu.sync_copy(rows_v, out_hbm.at[wid])

    return gather_kernel(x32, table)


def _tc_fused(e, mask_f, wp, bp, w1, b1, w2, b2, wmu, bmu, wlv, blv, eps):
    def body(e_ref, m_ref, wp_ref, bp_ref, w1_ref, b1_ref, w2_ref, b2_ref,
             wmu_ref, bmu_ref, wlv_ref, blv_ref, eps_ref,
             z_ref, mu_ref, lv_ref):
        hp = jax.lax.Precision.HIGHEST
        ev = e_ref[...]
        p = jnp.dot(ev, wp_ref[...], precision=hp) + bp_ref[...]
        p = jnp.maximum(p, 0.0)
        p = p.reshape(BB, L, K_DIM) * m_ref[...][:, :, None]
        pnc = jnp.sum(p, axis=1)
        h = jnp.maximum(jnp.dot(pnc, w1_ref[...], precision=hp) + b1_ref[...], 0.0)
        h = jnp.maximum(jnp.dot(h, w2_ref[...], precision=hp) + b2_ref[...], 0.0)
        mu = jnp.dot(h, wmu_ref[...], precision=hp) + bmu_ref[...]
        lv = jnp.dot(h, wlv_ref[...], precision=hp) + blv_ref[...]
        z = mu + eps_ref[...] * jnp.exp(0.5 * lv)
        z_ref[...] = z
        mu_ref[...] = mu
        lv_ref[...] = lv

    rep = lambda shape: pl.BlockSpec(shape, lambda i: (0,) * len(shape))
    out_sds = jax.ShapeDtypeStruct((B, LAT), jnp.float32)
    return pl.pallas_call(
        body,
        grid=(GRID,),
        in_specs=[
            pl.BlockSpec((BB * L, EMB), lambda i: (i, 0)),
            pl.BlockSpec((BB, L), lambda i: (i, 0)),
            rep((EMB, K_DIM)), rep((1, K_DIM)),
            rep((K_DIM, H1)), rep((1, H1)),
            rep((H1, H2)), rep((1, H2)),
            rep((H2, LAT)), rep((1, LAT)),
            rep((H2, LAT)), rep((1, LAT)),
            pl.BlockSpec((BB, LAT), lambda i: (i, 0)),
        ],
        out_specs=[
            pl.BlockSpec((BB, LAT), lambda i: (i, 0)),
            pl.BlockSpec((BB, LAT), lambda i: (i, 0)),
            pl.BlockSpec((BB, LAT), lambda i: (i, 0)),
        ],
        out_shape=[out_sds, out_sds, out_sds],
    )(e, mask_f, wp, bp, w1, b1, w2, b2, wmu, bmu, wlv, blv, eps)


def kernel(x, mask, table, W_pnnn, b_pnnn, W1, b1, W2, b2, Wmu, bmu, Wlv, blv, eps):
    x32 = x.reshape(NW, NCH, CHUNK)
    table_lin = _tc_pack(table.T).reshape(V, EMB)
    gathered = _sc_gather(x32, table_lin).reshape(TOT, EMB)
    mask_f = mask.astype(jnp.float32)
    z, mu, lv = _tc_fused(
        gathered, mask_f,
        W_pnnn.T, b_pnnn.reshape(1, K_DIM),
        W1.T, b1.reshape(1, H1),
        W2.T, b2.reshape(1, H2),
        Wmu.T, bmu.reshape(1, LAT),
        Wlv.T, blv.reshape(1, LAT),
        eps,
    )
    return (z, mu, lv)


# speedup vs baseline: 4.5102x; 1.1780x over previous
import functools

import jax
import jax.numpy as jnp
from jax import lax
from jax.experimental import pallas as pl
from jax.experimental.pallas import tpu as pltpu
from jax.experimental.pallas import tpu_sc as plsc

B, L = 1024, 100
EMB = 16
K_DIM, H1, H2, LAT = 64, 128, 64, 32

NC, NS = 2, 16
NW = NC * NS
TOT = B * L
PER_W = TOT // NW
CHUNK = 128
NCH = PER_W // CHUNK

BB = 64
GRID = B // BB

V = 1000000
CC = 8192


def _tc_pack(table_t):
    def body(i_ref, o_ref):
        t = i_ref[...].T
        t3 = t.reshape(CC // 8, 8, 16)
        o_ref[...] = jnp.concatenate([t3[:, j, :] for j in range(8)], axis=1)

    return pl.pallas_call(
        body,
        grid=((V + CC - 1) // CC,),
        in_specs=[pl.BlockSpec((16, CC), lambda i: (0, i))],
        out_specs=pl.BlockSpec((CC // 8, 128), lambda i: (i, 0)),
        out_shape=jax.ShapeDtypeStruct((V // 8, 128), jnp.float32),
    )(table_t)


def _sc_gather(x32, table):
    mesh = plsc.VectorSubcoreMesh(core_axis_name="c", subcore_axis_name="s")

    @functools.partial(
        pl.kernel,
        out_type=jax.ShapeDtypeStruct((NW, PER_W, EMB), jnp.float32),
        mesh=mesh,
        scratch_types=[
            pltpu.VMEM((NCH, CHUNK), jnp.int32),
            pltpu.VMEM((PER_W, EMB), jnp.float32),
            pltpu.SemaphoreType.DMA,
        ],
        compiler_params=pltpu.CompilerParams(use_tc_tiling_on_sc=False),
    )
    def gather_kernel(x_hbm, table_hbm, out_hbm, idx_v, rows_v, sem):
        wid = lax.axis_index("s") * NC + lax.axis_index("c")
        pltpu.sync_copy(x_hbm.at[wid], idx_v)

        def fire(j, _):
            pltpu.make_async_copy(
                table_hbm.at[idx_v.at[j]],
                rows_v.at[pl.ds(j * CHUNK, CHUNK)],
                sem,
            ).start()
            return _

        lax.fori_loop(0, NCH, fire, 0, unroll=False)

        def drain(j, _):
            pltpu.make_async_copy(
                table_hbm.at[idx_v.at[j]],
                rows_v.at[pl.ds(j * CHUNK, CHUNK)],
                sem,
            ).wait()
            return _

        lax.fori_loop(0, NCH, drain, 0, unroll=False)
        pltpu.sync_copy(rows_v, out_hbm.at[wid])

    return gather_kernel(x32, table)


RPB = BB * L // 8


def _tc_fused(e8, mjs3, wbig, bbig, w1, b1, w2, b2, wmu, bmu, wlv, blv, eps):
    def body(e_ref, m_ref, wb_ref, bb_ref, w1_ref, b1_ref, w2_ref, b2_ref,
             wmu_ref, bmu_ref, wlv_ref, blv_ref, eps_ref,
             z_ref, mu_ref, lv_ref):
        hp = jax.lax.Precision.HIGHEST
        p8 = jnp.dot(e_ref[...], wb_ref[...]) + bb_ref[...]
        p8 = jnp.maximum(p8, 0.0)
        bb_lo = jax.lax.broadcasted_iota(jnp.int32, (BB, RPB), 0) * L
        el8 = jax.lax.broadcasted_iota(jnp.int32, (BB, RPB), 1) * 8
        pnc = jnp.zeros((BB, K_DIM), jnp.float32)
        for j in range(8):
            el = el8 + j
            sel = ((el >= bb_lo) & (el < bb_lo + L)).astype(jnp.float32)
            sel = sel * m_ref[0, j, :][None, :]
            pnc = pnc + jnp.dot(sel, p8[:, 64 * j:64 * (j + 1)])
        h = jnp.maximum(jnp.dot(pnc, w1_ref[...], precision=hp) + b1_ref[...], 0.0)
        h = jnp.maximum(jnp.dot(h, w2_ref[...], precision=hp) + b2_ref[...], 0.0)
        mu = jnp.dot(h, wmu_ref[...], precision=hp) + bmu_ref[...]
        lv = jnp.dot(h, wlv_ref[...], precision=hp) + blv_ref[...]
        z = mu + eps_ref[...] * jnp.exp(0.5 * lv)
        z_ref[...] = z
        mu_ref[...] = mu
        lv_ref[...] = lv

    rep = lambda shape: pl.BlockSpec(shape, lambda i: (0,) * len(shape))
    out_sds = jax.ShapeDtypeStruct((B, LAT), jnp.float32)
    return pl.pallas_call(
        body,
        grid=(GRID,),
        in_specs=[
            pl.BlockSpec((RPB, 128), lambda i: (i, 0)),
            pl.BlockSpec((1, 8, RPB), lambda i: (i, 0, 0)),
            rep((128, 8 * K_DIM)), rep((1, 8 * K_DIM)),
            rep((K_DIM, H1)), rep((1, H1)),
            rep((H1, H2)), rep((1, H2)),
            rep((H2, LAT)), rep((1, LAT)),
            rep((H2, LAT)), rep((1, LAT)),
            pl.BlockSpec((BB, LAT), lambda i: (i, 0)),
        ],
        out_specs=[
            pl.BlockSpec((BB, LAT), lambda i: (i, 0)),
            pl.BlockSpec((BB, LAT), lambda i: (i, 0)),
            pl.BlockSpec((BB, LAT), lambda i: (i, 0)),
        ],
        out_shape=[out_sds, out_sds, out_sds],
    )(e8, mjs3, wbig, bbig, w1, b1, w2, b2, wmu, bmu, wlv, blv, eps)


def kernel(x, mask, table, W_pnnn, b_pnnn, W1, b1, W2, b2, Wmu, bmu, Wlv, blv, eps):
    x32 = x.reshape(NW, NCH, CHUNK)
    table_lin = _tc_pack(table.T).reshape(V, EMB)
    e8 = _sc_gather(x32, table_lin).reshape(TOT // 8, 128)
    mjs3 = mask.astype(jnp.float32).reshape(GRID, RPB, 8).transpose(0, 2, 1)
    wbig = jnp.kron(jnp.eye(8, dtype=jnp.float32), W_pnnn.T)
    bbig = jnp.tile(b_pnnn, 8).reshape(1, 8 * K_DIM)
    z, mu, lv = _tc_fused(
        e8, mjs3, wbig, bbig,
        W1.T, b1.reshape(1, H1),
        W2.T, b2.reshape(1, H2),
        Wmu.T, bmu.reshape(1, LAT),
        Wlv.T, blv.reshape(1, LAT),
        eps,
    )
    return (z, mu, lv)
